# baseline (speedup 1.0000x reference)
import jax
import jax.numpy as jnp
from jax.experimental import pallas as pl
from jax.experimental.pallas import tpu as pltpu

N = 4096
F = 64
BM = 256
NSTEPS = N // BM
NBUF = 2
NSPLIT = 2
SUB = N // NSPLIT


def _fused_kernel(b_hbm, adj_hbm, w_hbm, bias_hbm, out_ref,
                  buf_ref, sup_ref, b_ref, w_ref, bias_ref,
                  sem_ref, sem_small):
    i = pl.program_id(0)

    def _copy(block, slot, s):
        return pltpu.make_async_copy(
            adj_hbm.at[pl.ds(block * BM, BM), pl.ds(s * SUB, SUB)],
            buf_ref.at[slot, :, pl.ds(s * SUB, SUB)],
            sem_ref.at[slot, s],
        )

    def _start_block(block, slot):
        for s in range(NSPLIT):
            _copy(block, slot, s).start()

    def _wait_block(block, slot):
        for s in range(NSPLIT):
            _copy(block, slot, s).wait()

    @pl.when(i == 0)
    def _():
        for j in range(NBUF):
            _start_block(j, j)
        cb = pltpu.make_async_copy(b_hbm, b_ref, sem_small.at[0])
        cw = pltpu.make_async_copy(w_hbm, w_ref, sem_small.at[1])
        cs = pltpu.make_async_copy(bias_hbm, bias_ref.at[0], sem_small.at[2])
        cb.start()
        cw.start()
        cs.start()
        cb.wait()
        cw.wait()
        cs.wait()
        sup_ref[...] = jnp.dot(
            b_ref[...], w_ref[...], preferred_element_type=jnp.float32
        ).astype(jnp.bfloat16)

    slot = jax.lax.rem(i, NBUF)
    _wait_block(i, slot)
    adj_bf = buf_ref[slot].astype(jnp.bfloat16)
    out_ref[...] = (
        jnp.dot(adj_bf, sup_ref[...], preferred_element_type=jnp.float32)
        + bias_ref[...]
    )

    nxt = i + NBUF

    @pl.when(nxt < NSTEPS)
    def _():
        _start_block(nxt, slot)


def kernel(b_input, adj, a_weight, a_bias):
    return pl.pallas_call(
        _fused_kernel,
        grid=(NSTEPS,),
        in_specs=[
            pl.BlockSpec(memory_space=pltpu.MemorySpace.HBM),
            pl.BlockSpec(memory_space=pltpu.MemorySpace.HBM),
            pl.BlockSpec(memory_space=pltpu.MemorySpace.HBM),
            pl.BlockSpec(memory_space=pltpu.MemorySpace.HBM),
        ],
        out_specs=pl.BlockSpec((BM, F), lambda i: (i, 0)),
        out_shape=jax.ShapeDtypeStruct((N, F), jnp.float32),
        scratch_shapes=[
            pltpu.VMEM((NBUF, BM, N), jnp.float32),
            pltpu.VMEM((N, F), jnp.bfloat16),
            pltpu.VMEM((N, F), jnp.float32),
            pltpu.VMEM((F, F), jnp.float32),
            pltpu.VMEM((1, F), jnp.float32),
            pltpu.SemaphoreType.DMA((NBUF, NSPLIT)),
            pltpu.SemaphoreType.DMA((3,)),
        ],
    )(b_input, adj, a_weight, a_bias)


# speedup vs baseline: 1.0765x; 1.0765x over previous
import jax
import jax.numpy as jnp
from jax.experimental import pallas as pl
from jax.experimental.pallas import tpu as pltpu

N = 4096
F = 64
BM = 512


def _fused_kernel(b_ref, adj_ref, w_ref, bias_ref, out_ref, sup_ref):
    @pl.when(pl.program_id(0) == 0)
    def _():
        sup_ref[...] = jnp.dot(
            b_ref[...], w_ref[...], preferred_element_type=jnp.float32
        ).astype(jnp.bfloat16)

    adj_bf = adj_ref[...].astype(jnp.bfloat16)
    out_ref[...] = (
        jnp.dot(adj_bf, sup_ref[...], preferred_element_type=jnp.float32)
        + bias_ref[...]
    )


def kernel(b_input, adj, a_weight, a_bias):
    bias2d = a_bias.reshape(1, F)
    grid = (N // BM,)
    return pl.pallas_call(
        _fused_kernel,
        grid=grid,
        in_specs=[
            pl.BlockSpec((N, F), lambda i: (0, 0)),
            pl.BlockSpec((BM, N), lambda i: (i, 0)),
            pl.BlockSpec((F, F), lambda i: (0, 0)),
            pl.BlockSpec((1, F), lambda i: (0, 0)),
        ],
        out_specs=pl.BlockSpec((BM, F), lambda i: (i, 0)),
        out_shape=jax.ShapeDtypeStruct((N, F), jnp.float32),
        scratch_shapes=[pltpu.VMEM((N, F), jnp.bfloat16)],
    )(b_input, adj, a_weight, bias2d)


# speedup vs baseline: 1.2758x; 1.1852x over previous
import jax
import jax.numpy as jnp
from jax.experimental import pallas as pl
from jax.experimental.pallas import tpu as pltpu

N = 4096
F = 64
BM = 512


def _fused_kernel(b_ref, adj_ref, w_ref, bias_ref, out_ref, sup_ref):
    @pl.when(pl.program_id(0) == 0)
    def _():
        sup_ref[...] = jnp.dot(
            b_ref[...], w_ref[...], preferred_element_type=jnp.float32
        ).astype(jnp.bfloat16)

    adj_bf = adj_ref[...].astype(jnp.bfloat16)
    out_ref[...] = (
        jnp.dot(adj_bf, sup_ref[...], preferred_element_type=jnp.float32)
        + bias_ref[...]
    )


def kernel(b_input, adj, a_weight, a_bias):
    bias2d = a_bias.reshape(1, F)
    grid = (N // BM,)
    return pl.pallas_call(
        _fused_kernel,
        grid=grid,
        in_specs=[
            pl.BlockSpec((N, F), lambda i: (0, 0)),
            pl.BlockSpec((BM, N), lambda i: (i, 0)),
            pl.BlockSpec((F, F), lambda i: (0, 0)),
            pl.BlockSpec((1, F), lambda i: (0, 0)),
        ],
        out_specs=pl.BlockSpec((BM, F), lambda i: (i, 0)),
        out_shape=jax.ShapeDtypeStruct((N, F), jnp.float32),
        scratch_shapes=[pltpu.VMEM((N, F), jnp.bfloat16)],
        compiler_params=pltpu.CompilerParams(
            allow_input_fusion=[True, True, True, True],
        ),
    )(b_input, adj, a_weight, bias2d)


# speedup vs baseline: 1.2885x; 1.0099x over previous
import jax
import jax.numpy as jnp
from jax.experimental import pallas as pl
from jax.experimental.pallas import tpu as pltpu

N = 4096
F = 64
BM = 512


def _fused_kernel(b_ref, adj_ref, w_ref, bias_ref, out_ref, sup_ref):
    @pl.when(pl.program_id(0) == 0)
    def _():
        sup_ref[...] = jnp.dot(
            b_ref[...], w_ref[...], preferred_element_type=jnp.float32
        ).astype(jnp.bfloat16)

    adj_bf = adj_ref[...].astype(jnp.bfloat16)
    res = (
        jnp.dot(adj_bf, sup_ref[...], preferred_element_type=jnp.float32)
        + bias_ref[...]
    )
    out_ref[...] = res.astype(jnp.bfloat16)


def kernel(b_input, adj, a_weight, a_bias):
    bias2d = a_bias.reshape(1, F)
    b16 = b_input.astype(jnp.bfloat16)
    w16 = a_weight.astype(jnp.bfloat16)
    grid = (N // BM,)
    out16 = pl.pallas_call(
        _fused_kernel,
        grid=grid,
        in_specs=[
            pl.BlockSpec((N, F), lambda i: (0, 0)),
            pl.BlockSpec((BM, N), lambda i: (i, 0)),
            pl.BlockSpec((F, F), lambda i: (0, 0)),
            pl.BlockSpec((1, F), lambda i: (0, 0)),
        ],
        out_specs=pl.BlockSpec((BM, F), lambda i: (i, 0)),
        out_shape=jax.ShapeDtypeStruct((N, F), jnp.bfloat16),
        scratch_shapes=[pltpu.VMEM((N, F), jnp.bfloat16)],
        compiler_params=pltpu.CompilerParams(
            allow_input_fusion=[True, True, True, True],
        ),
    )(b16, adj, w16, bias2d)
    return out16.astype(jnp.float32)
